# baseline (speedup 1.0000x reference)
import functools

import jax
import jax.numpy as jnp
from jax import lax
from jax.experimental import pallas as pl
from jax.experimental.pallas import tpu as pltpu
from jax.experimental.pallas import tpu_sc as plsc

B = 1024
N = 100000
K = 200
NC = 2
NS = 16
NW = NC * NS
TOT = B * N
ELEMS_PER_W = TOT // NW
CHUNK = 128
NCHUNK = (B * K) // CHUNK
CH_PER_W = NCHUNK // NW
SUB = CHUNK // 16


def _scatter_body(x_hbm, idx_hbm, src_hbm, out_hbm, idx_v, src_v, flat_v):
    c = lax.axis_index("c")
    s = lax.axis_index("s")
    w = s * NC + c

    base = w * ELEMS_PER_W
    pltpu.sync_copy(x_hbm.at[pl.ds(base, ELEMS_PER_W)],
                    out_hbm.at[pl.ds(base, ELEMS_PER_W)])

    cb = w * CH_PER_W
    pltpu.sync_copy(idx_hbm.at[w], idx_v)
    pltpu.sync_copy(src_hbm.at[w], src_v)

    def idx_body(j, carry):
        for k in range(SUB):
            t0 = (cb + j) * CHUNK + k * 16
            r0 = t0 // K
            rem = t0 - r0 * K
            lane = lax.iota(jnp.int32, 16)
            bump = jnp.where(lane + rem >= K, jnp.int32(N), jnp.int32(0))
            flat_v[j, k * 16:(k + 1) * 16] = (
                idx_v[j, k * 16:(k + 1) * 16] + (r0 * N + bump))
        return carry

    lax.fori_loop(0, CH_PER_W, idx_body, 0)

    def sc_body(j, carry):
        pltpu.sync_copy(src_v.at[j], out_hbm.at[flat_v.at[j]])
        return carry

    lax.fori_loop(0, CH_PER_W, sc_body, 0)


@jax.jit
def _scatter_op(x_flat, idx_2d, src_2d):
    mesh = plsc.VectorSubcoreMesh(core_axis_name="c", subcore_axis_name="s")
    run = pl.kernel(
        _scatter_body,
        out_type=jax.ShapeDtypeStruct((TOT,), jnp.float32),
        mesh=mesh,
        scratch_types=[
            pltpu.VMEM((CH_PER_W, CHUNK), jnp.int32),
            pltpu.VMEM((CH_PER_W, CHUNK), jnp.float32),
            pltpu.VMEM((CH_PER_W, CHUNK), jnp.int32),
        ],
    )
    return run(x_flat, idx_2d, src_2d)


def kernel(x, indices, src):
    x_flat = jnp.reshape(x, (TOT,))
    idx_2d = jnp.reshape(indices.astype(jnp.int32), (NW, CH_PER_W, CHUNK))
    src_2d = jnp.reshape(src.astype(jnp.float32), (NW, CH_PER_W, CHUNK))
    out_flat = _scatter_op(x_flat, idx_2d, src_2d)
    return jnp.reshape(out_flat, (B, N))


# speedup vs baseline: 6.4370x; 6.4370x over previous
import functools

import jax
import jax.numpy as jnp
from jax import lax
from jax.experimental import pallas as pl
from jax.experimental.pallas import tpu as pltpu
from jax.experimental.pallas import tpu_sc as plsc

B = 1024
N = 100000
K = 200
NC = 2
NS = 16
NW = NC * NS
TOT = B * N
ELEMS_PER_W = TOT // NW
CP = 50_000
NCH = ELEMS_PER_W // CP
CHUNK = 128
NCHUNK = (B * K) // CHUNK
CH_PER_W = NCHUNK // NW
SUB = CHUNK // 16


def _scatter_body(x_hbm, idx_hbm, src_hbm, out_hbm,
                  idx_v, src_v, flat_v, buf0, buf1, sem0, sem1):
    c = lax.axis_index("c")
    s = lax.axis_index("s")
    w = s * NC + c
    base = w * ELEMS_PER_W

    def in_sl(i):
        return x_hbm.at[pl.ds(base + i * CP, CP)]

    def out_sl(i):
        return out_hbm.at[pl.ds(base + i * CP, CP)]

    pltpu.async_copy(in_sl(0), buf0, sem0)

    pltpu.sync_copy(idx_hbm.at[w], idx_v)
    pltpu.sync_copy(src_hbm.at[w], src_v)

    cb = w * CH_PER_W

    def idx_body(j, carry):
        for k in range(SUB):
            t0 = (cb + j) * CHUNK + k * 16
            r0 = t0 // K
            rem = t0 - r0 * K
            lane = lax.iota(jnp.int32, 16)
            bump = jnp.where(lane + rem >= K, jnp.int32(N), jnp.int32(0))
            flat_v[j, k * 16:(k + 1) * 16] = (
                idx_v[j, k * 16:(k + 1) * 16] + (r0 * N + bump))
        return carry

    lax.fori_loop(0, CH_PER_W, idx_body, 0)

    def copy_body(it, carry):
        i0 = 2 * it
        pltpu.make_async_copy(in_sl(i0), buf0, sem0).wait()
        pltpu.async_copy(in_sl(i0 + 1), buf1, sem1)
        pltpu.sync_copy(buf0, out_sl(i0))
        pltpu.make_async_copy(in_sl(i0 + 1), buf1, sem1).wait()

        @pl.when(it < NCH // 2 - 1)
        def _():
            pltpu.async_copy(in_sl(i0 + 2), buf0, sem0)

        pltpu.sync_copy(buf1, out_sl(i0 + 1))
        return carry

    lax.fori_loop(0, NCH // 2, copy_body, 0)

    def sc_body(j, carry):
        pltpu.sync_copy(src_v.at[j], out_hbm.at[flat_v.at[j]])
        return carry

    lax.fori_loop(0, CH_PER_W, sc_body, 0)


@jax.jit
def _scatter_op(x_flat, idx_2d, src_2d):
    mesh = plsc.VectorSubcoreMesh(core_axis_name="c", subcore_axis_name="s")
    run = pl.kernel(
        _scatter_body,
        out_type=jax.ShapeDtypeStruct((TOT,), jnp.float32),
        mesh=mesh,
        scratch_types=[
            pltpu.VMEM((CH_PER_W, CHUNK), jnp.int32),
            pltpu.VMEM((CH_PER_W, CHUNK), jnp.float32),
            pltpu.VMEM((CH_PER_W, CHUNK), jnp.int32),
            pltpu.VMEM((CP,), jnp.float32),
            pltpu.VMEM((CP,), jnp.float32),
            pltpu.SemaphoreType.DMA,
            pltpu.SemaphoreType.DMA,
        ],
    )
    return run(x_flat, idx_2d, src_2d)


def kernel(x, indices, src):
    x_flat = jnp.reshape(x, (TOT,))
    idx_2d = jnp.reshape(indices.astype(jnp.int32), (NW, CH_PER_W, CHUNK))
    src_2d = jnp.reshape(src.astype(jnp.float32), (NW, CH_PER_W, CHUNK))
    out_flat = _scatter_op(x_flat, idx_2d, src_2d)
    return jnp.reshape(out_flat, (B, N))
